# Initial kernel scaffold; baseline (speedup 1.0000x reference)
import functools

import jax
import jax.numpy as jnp
from jax import lax
from jax.experimental import pallas as pl
from jax.experimental.pallas import tpu as pltpu
from jax.experimental.pallas import tpu_sc as plsc

BATCH = 4096
SEQ = 50
EMBED = 64
HIDDEN = 128
LABELS = 32

NC = 2
NS = 16
NW = NC * NS
ROWS_PER_W = BATCH // NW
ROWS_PER_CHUNK = 2
IDS_PER_CHUNK = ROWS_PER_CHUNK * SEQ
CHUNKS = ROWS_PER_W // ROWS_PER_CHUNK


def _sc_pool_body(ids_hbm, table_hbm, out_hbm, idx_v, rows_v, out_v, sem):
    c = lax.axis_index("c")
    s = lax.axis_index("s")
    wid = s * NC + c
    base = wid * CHUNKS
    pltpu.sync_copy(ids_hbm.at[pl.ds(base, CHUNKS)], idx_v)

    def chunk(g, carry):
        pltpu.async_copy(table_hbm.at[idx_v.at[g]], rows_v, sem).wait()
        for r in range(ROWS_PER_CHUNK):
            for cc in range(EMBED // 16):
                sl = pl.ds(cc * 16, 16)
                acc = rows_v[SEQ * r, sl]
                for t in range(1, SEQ):
                    acc = acc + rows_v[SEQ * r + t, sl]
            out_v[r, sl] = acc
        pltpu.sync_copy(
            out_v, out_hbm.at[pl.ds(wid * ROWS_PER_W + g * ROWS_PER_CHUNK,
                                    ROWS_PER_CHUNK)])
        return carry

    lax.fori_loop(0, CHUNKS, chunk, 0)


_sc_pool = functools.partial(
    pl.kernel,
    out_type=jax.ShapeDtypeStruct((BATCH, EMBED), jnp.float32),
    mesh=plsc.VectorSubcoreMesh(core_axis_name="c", subcore_axis_name="s"),
    scratch_types=[
        pltpu.VMEM((CHUNKS, IDS_PER_CHUNK), jnp.int32),
        pltpu.VMEM((IDS_PER_CHUNK, EMBED), jnp.float32),
        pltpu.VMEM((ROWS_PER_CHUNK, EMBED), jnp.float32),
        pltpu.SemaphoreType.DMA,
    ],
)(_sc_pool_body)


def _tc_head_body(ids_ref, psum_ref, gamma_ref, beta_ref, w1_ref, b1_ref,
                  w2_ref, b2_ref, out_ref):
    ids = ids_ref[...]
    cnt = jnp.sum((ids != 0).astype(jnp.float32), axis=1, keepdims=True)
    pooled = psum_ref[...] / jnp.maximum(cnt, 1.0)
    mean = jnp.mean(pooled, axis=1, keepdims=True)
    centered = pooled - mean
    var = jnp.mean(centered * centered, axis=1, keepdims=True)
    normed = centered * lax.rsqrt(var + 1e-5) * gamma_ref[...] + beta_ref[...]
    h = jnp.dot(normed, w1_ref[...], preferred_element_type=jnp.float32)
    h = jnp.maximum(h + b1_ref[...], 0.0)
    out = jnp.dot(h, w2_ref[...], preferred_element_type=jnp.float32)
    out_ref[...] = out + b2_ref[...]


def _tc_head(token_ids, psum, gamma, beta, W1, b1, W2, b2):
    blk = 512
    grid = BATCH // blk
    return pl.pallas_call(
        _tc_head_body,
        grid=(grid,),
        in_specs=[
            pl.BlockSpec((blk, SEQ), lambda i: (i, 0)),
            pl.BlockSpec((blk, EMBED), lambda i: (i, 0)),
            pl.BlockSpec((1, EMBED), lambda i: (0, 0)),
            pl.BlockSpec((1, EMBED), lambda i: (0, 0)),
            pl.BlockSpec((EMBED, HIDDEN), lambda i: (0, 0)),
            pl.BlockSpec((1, HIDDEN), lambda i: (0, 0)),
            pl.BlockSpec((HIDDEN, LABELS), lambda i: (0, 0)),
            pl.BlockSpec((1, LABELS), lambda i: (0, 0)),
        ],
        out_specs=pl.BlockSpec((blk, LABELS), lambda i: (i, 0)),
        out_shape=jax.ShapeDtypeStruct((BATCH, LABELS), jnp.float32),
    )(token_ids, psum, gamma, beta, W1, b1, W2, b2)


def kernel(token_ids, table, gamma, beta, W1, b1, W2, b2):
    ids2d = token_ids.astype(jnp.int32).reshape(-1, IDS_PER_CHUNK)
    psum = _sc_pool(ids2d, table)
    return _tc_head(token_ids.astype(jnp.int32), psum,
                    gamma.reshape(1, EMBED), beta.reshape(1, EMBED),
                    W1, b1.reshape(1, HIDDEN), W2, b2.reshape(1, LABELS))


# speedup vs baseline: 5.9366x; 5.9366x over previous
import functools

import jax
import jax.numpy as jnp
from jax import lax
from jax.experimental import pallas as pl
from jax.experimental.pallas import tpu as pltpu
from jax.experimental.pallas import tpu_sc as plsc

BATCH = 4096
SEQ = 50
EMBED = 64
HIDDEN = 128
LABELS = 32

NC = 2
NS = 16
NW = NC * NS
ROWS_PER_W = BATCH // NW
ROWS_PER_CHUNK = 2
IDS_PER_CHUNK = ROWS_PER_CHUNK * SEQ
CHUNKS = ROWS_PER_W // ROWS_PER_CHUNK


def _sc_pool_body(ids_hbm, table_hbm, out_hbm, idx_v, rows_v, out_v, sem):
    c = lax.axis_index("c")
    s = lax.axis_index("s")
    wid = s * NC + c
    base = wid * CHUNKS
    pltpu.sync_copy(ids_hbm.at[pl.ds(base, CHUNKS)], idx_v)

    def chunk(g, carry):
        pltpu.async_copy(table_hbm.at[idx_v.at[g]], rows_v, sem).wait()
        for r in range(ROWS_PER_CHUNK):
            for cc in range(EMBED // 16):
                sl = pl.ds(cc * 16, 16)
                acc = rows_v[SEQ * r, sl]
                for t in range(1, SEQ):
                    acc = acc + rows_v[SEQ * r + t, sl]
                out_v[r, sl] = acc
        pltpu.sync_copy(
            out_v, out_hbm.at[pl.ds(wid * ROWS_PER_W + g * ROWS_PER_CHUNK,
                                    ROWS_PER_CHUNK)])
        return carry

    lax.fori_loop(0, CHUNKS, chunk, 0)


_sc_pool = functools.partial(
    pl.kernel,
    out_type=jax.ShapeDtypeStruct((BATCH, EMBED), jnp.float32),
    mesh=plsc.VectorSubcoreMesh(core_axis_name="c", subcore_axis_name="s"),
    scratch_types=[
        pltpu.VMEM((CHUNKS, IDS_PER_CHUNK), jnp.int32),
        pltpu.VMEM((IDS_PER_CHUNK, EMBED), jnp.float32),
        pltpu.VMEM((ROWS_PER_CHUNK, EMBED), jnp.float32),
        pltpu.SemaphoreType.DMA,
    ],
    compiler_params=pltpu.CompilerParams(use_tc_tiling_on_sc=False),
)(_sc_pool_body)


def _tc_head_body(ids_ref, psum_ref, gamma_ref, beta_ref, w1_ref, b1_ref,
                  w2_ref, b2_ref, out_ref):
    ids = ids_ref[...]
    cnt = jnp.sum((ids != 0).astype(jnp.float32), axis=1, keepdims=True)
    pooled = psum_ref[...] / jnp.maximum(cnt, 1.0)
    mean = jnp.mean(pooled, axis=1, keepdims=True)
    centered = pooled - mean
    var = jnp.mean(centered * centered, axis=1, keepdims=True)
    normed = centered * lax.rsqrt(var + 1e-5) * gamma_ref[...] + beta_ref[...]
    h = jnp.dot(normed, w1_ref[...], preferred_element_type=jnp.float32)
    h = jnp.maximum(h + b1_ref[...], 0.0)
    out = jnp.dot(h, w2_ref[...], preferred_element_type=jnp.float32)
    out_ref[...] = out + b2_ref[...]


def _tc_head(token_ids, psum, gamma, beta, W1, b1, W2, b2):
    blk = 512
    grid = BATCH // blk
    return pl.pallas_call(
        _tc_head_body,
        grid=(grid,),
        in_specs=[
            pl.BlockSpec((blk, SEQ), lambda i: (i, 0)),
            pl.BlockSpec((blk, EMBED), lambda i: (i, 0)),
            pl.BlockSpec((1, EMBED), lambda i: (0, 0)),
            pl.BlockSpec((1, EMBED), lambda i: (0, 0)),
            pl.BlockSpec((EMBED, HIDDEN), lambda i: (0, 0)),
            pl.BlockSpec((1, HIDDEN), lambda i: (0, 0)),
            pl.BlockSpec((HIDDEN, LABELS), lambda i: (0, 0)),
            pl.BlockSpec((1, LABELS), lambda i: (0, 0)),
        ],
        out_specs=pl.BlockSpec((blk, LABELS), lambda i: (i, 0)),
        out_shape=jax.ShapeDtypeStruct((BATCH, LABELS), jnp.float32),
    )(token_ids, psum, gamma, beta, W1, b1, W2, b2)


def kernel(token_ids, table, gamma, beta, W1, b1, W2, b2):
    ids2d = token_ids.astype(jnp.int32).reshape(-1, IDS_PER_CHUNK)
    psum = _sc_pool(ids2d, table)
    return _tc_head(token_ids.astype(jnp.int32), psum,
                    gamma.reshape(1, EMBED), beta.reshape(1, EMBED),
                    W1, b1.reshape(1, HIDDEN), W2, b2.reshape(1, LABELS))
